# Initial kernel scaffold; baseline (speedup 1.0000x reference)
import jax
import jax.numpy as jnp
from jax.experimental import pallas as pl


def kernel(obs, action, W0_q1, b0_q1, W1_q1, b1_q1, W2_q1, b2_q1, W0_q2, b0_q2, W1_q2, b1_q2, W2_q2, b2_q2):
    raise NotImplementedError("write your pallas kernel here")


# speedup vs baseline: 116.3922x; 116.3922x over previous
import jax
import jax.numpy as jnp
from jax.experimental import pallas as pl

_NODES = 8
_F = 128
_TILE = 1024


def _mix(h):
    p = jnp.concatenate([h[:, -_F:], h[:, :-_F]], axis=1)
    m = jnp.concatenate([h[:, _F:], h[:, :_F]], axis=1)
    return (h + p + m) * (1.0 / 3.0)


def _head(x, w0k, w1, w2k, b0, b1, b2):
    h = jax.lax.dot(x, w0k, preferred_element_type=jnp.float32) + b0
    h = jnp.maximum(h, 0.0)
    hm = _mix(h)
    cols = [
        jax.lax.dot(hm[:, n * _F:(n + 1) * _F], w1,
                    preferred_element_type=jnp.float32)
        for n in range(_NODES)
    ]
    h = jnp.concatenate(cols, axis=1) + b1
    h = jnp.maximum(h, 0.0)
    return jax.lax.dot(h, w2k, preferred_element_type=jnp.float32) + b2


def _body(x_ref,
          w0a_ref, w1a_ref, w2a_ref, b0a_ref, b1a_ref, b2a_ref,
          w0b_ref, w1b_ref, w2b_ref, b0b_ref, b1b_ref, b2b_ref,
          q1_ref, q2_ref):
    x = x_ref[:, :]
    q1_ref[:, :] = _head(x, w0a_ref[:, :], w1a_ref[:, :], w2a_ref[:, :],
                         b0a_ref[:, :], b1a_ref[:, :], b2a_ref[:, :])
    q2_ref[:, :] = _head(x, w0b_ref[:, :], w1b_ref[:, :], w2b_ref[:, :],
                         b0b_ref[:, :], b1b_ref[:, :], b2b_ref[:, :])


def kernel(obs, action, W0_q1, b0_q1, W1_q1, b1_q1, W2_q1, b2_q1,
           W0_q2, b0_q2, W1_q2, b1_q2, W2_q2, b2_q2):
    bs = obs.shape[0]
    nodes = _NODES
    oa = jnp.concatenate(
        [obs.reshape(bs, nodes, -1), action.reshape(bs, nodes, -1)], axis=-1)
    in_dim = oa.shape[-1]
    x = oa.reshape(bs, nodes * in_dim)

    eye = jnp.eye(nodes, dtype=jnp.float32)
    amat = (eye + jnp.roll(eye, 1, axis=0) + jnp.roll(eye, -1, axis=0)) / 3.0

    def fold(w0, b0, w1, b1, w2, b2):
        return (jnp.kron(amat, w0),
                w1,
                jnp.kron(amat, w2),
                jnp.tile(b0, nodes)[None, :],
                jnp.tile(b1, nodes)[None, :],
                jnp.broadcast_to(b2, (nodes,))[None, :])

    p1 = fold(W0_q1, b0_q1, W1_q1, b1_q1, W2_q1, b2_q1)
    p2 = fold(W0_q2, b0_q2, W1_q2, b1_q2, W2_q2, b2_q2)

    tile = _TILE
    grid = (bs // tile,)
    full = lambda a: pl.BlockSpec(a.shape, lambda i: (0,) * a.ndim)
    q1, q2 = pl.pallas_call(
        _body,
        grid=grid,
        in_specs=[pl.BlockSpec((tile, x.shape[1]), lambda i: (i, 0))]
        + [full(a) for a in p1] + [full(a) for a in p2],
        out_specs=[pl.BlockSpec((tile, nodes), lambda i: (i, 0))] * 2,
        out_shape=[jax.ShapeDtypeStruct((bs, nodes), jnp.float32)] * 2,
    )(x, *p1, *p2)
    return (q1, q2)
